# baseline (speedup 1.0000x reference)
import functools

import jax
import jax.numpy as jnp
from jax import lax
from jax.experimental import pallas as pl
from jax.experimental.pallas import tpu as pltpu
from jax.experimental.pallas import tpu_sc as plsc

N = 10000
D = 128
H = 16
NPAD = 10240
E = 320000
GSZ = 128
NW = 16
G = 160
EPW = G * GSZ
EPAD = NW * EPW
RPT = NPAD // 16
GRID = NPAD // RPT

_MESH = plsc.VectorSubcoreMesh(core_axis_name="c", subcore_axis_name="s",
                               num_cores=1)


def _agg_body(with_counts, CB, NC, *args):
    if with_counts:
        (p, src_hbm, dst_hbm, zeros_hbm, ones_hbm,
         out_hbm, cnt_hbm,
         src_v, dst_v, rows_v, ones_v, acc_sh, stage_sh, cacc_sh, sem) = args
    else:
        (p, src_hbm, dst_hbm, zeros_hbm,
         out_hbm,
         src_v, dst_v, rows_v, acc_sh, stage_sh, sem) = args
    s = lax.axis_index("s")
    wid = s
    r0 = s * RPT
    pltpu.sync_copy(p.at[pl.ds(r0, RPT)], stage_sh.at[pl.ds(r0, RPT)])
    pltpu.sync_copy(zeros_hbm.at[pl.ds(r0, RPT)], acc_sh.at[pl.ds(r0, RPT)])
    if with_counts:
        pltpu.sync_copy(zeros_hbm.at[pl.ds(r0, RPT)], cacc_sh.at[pl.ds(r0, RPT)])
        pltpu.sync_copy(ones_hbm, ones_v)
    pltpu.sync_copy(src_hbm.at[wid], src_v)
    pltpu.sync_copy(dst_hbm.at[wid], dst_v)
    plsc.subcore_barrier()

    def chunk(ck, carry):
        gathers = []
        for b in range(CB):
            gathers.append((b, pltpu.async_copy(
                stage_sh.at[src_v.at[ck * CB + b]],
                rows_v.at[b], sem.at[b])))
        scatters = []
        for b, desc in gathers:
            desc.wait()
            g = ck * CB + b
            scatters.append(pltpu.async_copy(
                rows_v.at[b], acc_sh.at[dst_v.at[g]], sem.at[b], add=True))
            if with_counts:
                scatters.append(pltpu.async_copy(
                    ones_v, cacc_sh.at[dst_v.at[g]], sem.at[b], add=True))
        for desc in scatters:
            desc.wait()
        return carry

    lax.fori_loop(0, NC, chunk, 0)
    plsc.subcore_barrier()
    pltpu.sync_copy(acc_sh.at[pl.ds(r0, RPT)], out_hbm.at[pl.ds(r0, RPT)])
    if with_counts:
        pltpu.sync_copy(cacc_sh.at[pl.ds(r0, RPT)], cnt_hbm.at[pl.ds(r0, RPT)])


def _make_agg(with_counts):
    CB = 16
    NC = G // CB
    acc_t = jax.ShapeDtypeStruct((NPAD, H), jnp.float32)
    cnt_t = jax.ShapeDtypeStruct((NPAD, H), jnp.float32)
    out_types = [acc_t, cnt_t] if with_counts else [acc_t]
    scratch = [
        pltpu.VMEM((G, GSZ), jnp.int32),
        pltpu.VMEM((G, GSZ), jnp.int32),
        pltpu.VMEM((CB, GSZ, H), jnp.float32),
    ]
    if with_counts:
        scratch.append(pltpu.VMEM((GSZ, H), jnp.float32))
    scratch += [pltpu.VMEM_SHARED((NPAD, H), jnp.float32)] * 2
    if with_counts:
        scratch.append(pltpu.VMEM_SHARED((NPAD, H), jnp.float32))
    scratch.append(pltpu.SemaphoreType.DMA((CB,)))
    return pl.kernel(
        functools.partial(_agg_body, with_counts, CB, NC),
        out_type=tuple(out_types) if with_counts else out_types[0],
        mesh=_MESH,
        scratch_types=scratch,
        compiler_params=pltpu.CompilerParams(use_tc_tiling_on_sc=False),
    )


def _proj0_body(x1, x2, w1, w2, o1, o2):
    o1[...] = jnp.dot(x1[...], w1[...], preferred_element_type=jnp.float32)
    o2[...] = jnp.dot(x2[...], w2[...], preferred_element_type=jnp.float32)


def _proj0(x1, x2, w1, w2):
    return pl.pallas_call(
        _proj0_body,
        grid=(GRID,),
        in_specs=[
            pl.BlockSpec((RPT, D), lambda i: (i, 0)),
            pl.BlockSpec((RPT, D), lambda i: (i, 0)),
            pl.BlockSpec((D, H), lambda i: (0, 0)),
            pl.BlockSpec((D, H), lambda i: (0, 0)),
        ],
        out_specs=[pl.BlockSpec((RPT, H), lambda i: (i, 0))] * 2,
        out_shape=[jax.ShapeDtypeStruct((NPAD, H), jnp.float32)] * 2,
    )(x1, x2, w1, w2)


def _sage_out(acc, cnt, x, wr, b):
    agg = acc[...] / jnp.maximum(cnt[...], 1.0)
    return agg + jnp.dot(x[...], wr[...], preferred_element_type=jnp.float32) + b[...]


def _combine_body(accT, accS, cntT, cntS, xT, xS, wr1, b1, wr2, b2,
                  wl1n, wl2n, outT, outS, pnS, pnT):
    oT = _sage_out(accT, cntT, xT, wr1, b1)
    outT[...] = oT
    pnT[...] = jnp.dot(oT, wl2n[...], preferred_element_type=jnp.float32)
    oS = _sage_out(accS, cntS, xS, wr2, b2)
    outS[...] = oS
    pnS[...] = jnp.dot(oS, wl1n[...], preferred_element_type=jnp.float32)


def _make_combine(din):
    acc_spec = pl.BlockSpec((RPT, H), lambda i: (i, 0))
    cnt_spec = pl.BlockSpec((RPT, H), lambda i: (i, 0))
    w_spec = pl.BlockSpec((din, H), lambda i: (0, 0))
    b_spec = pl.BlockSpec((1, H), lambda i: (0, 0))
    h_spec = pl.BlockSpec((H, H), lambda i: (0, 0))
    row_spec = pl.BlockSpec((RPT, H), lambda i: (i, 0))
    return pl.pallas_call(
        _combine_body,
        grid=(GRID,),
        in_specs=[acc_spec, acc_spec, cnt_spec, cnt_spec,
                  pl.BlockSpec((RPT, din), lambda i: (i, 0)),
                  pl.BlockSpec((RPT, din), lambda i: (i, 0)),
                  w_spec, b_spec, w_spec, b_spec, h_spec, h_spec],
        out_specs=[row_spec] * 4,
        out_shape=[jax.ShapeDtypeStruct((NPAD, H), jnp.float32)] * 4,
    )


def _final_body(accT, accS, cntT, cntS, xT, xS, wr1, b1, wr2, b2,
                lw, lb, out, accum):
    i = pl.program_id(0)

    @pl.when(i == 0)
    def _():
        accum[...] = jnp.zeros_like(accum)

    oT = _sage_out(accT, cntT, xT, wr1, b1)
    oS = _sage_out(accS, cntS, xS, wr2, b2)
    rows = i * RPT + lax.broadcasted_iota(jnp.int32, (RPT, 1), 0)
    sblk = jnp.sum(jnp.where(rows < N, oT + oS, 0.0), axis=0, keepdims=True)
    accum[...] += sblk

    @pl.when(i == GRID - 1)
    def _():
        out[...] = (jnp.sum(accum[...] * lw[...], axis=1, keepdims=True)
                    / (2.0 * N) + lb[...])


def _make_final(din):
    acc_spec = pl.BlockSpec((RPT, H), lambda i: (i, 0))
    cnt_spec = pl.BlockSpec((RPT, H), lambda i: (i, 0))
    w_spec = pl.BlockSpec((din, H), lambda i: (0, 0))
    b_spec = pl.BlockSpec((1, H), lambda i: (0, 0))
    return pl.pallas_call(
        _final_body,
        grid=(GRID,),
        in_specs=[acc_spec, acc_spec, cnt_spec, cnt_spec,
                  pl.BlockSpec((RPT, din), lambda i: (i, 0)),
                  pl.BlockSpec((RPT, din), lambda i: (i, 0)),
                  w_spec, b_spec, w_spec, b_spec,
                  pl.BlockSpec((1, H), lambda i: (0, 0)),
                  pl.BlockSpec((1, 1), lambda i: (0, 0))],
        out_specs=pl.BlockSpec((1, 1), lambda i: (0, 0)),
        out_shape=jax.ShapeDtypeStruct((1, 1), jnp.float32),
        scratch_shapes=[pltpu.VMEM((1, H), jnp.float32)],
    )


def kernel(x_source, x_target, edge_index_s2t, edge_index_t2s,
           edge_attr_s2t, edge_attr_t2s, params_s2t, params_t2s,
           lin_W, lin_b):
    f32 = jnp.float32
    xs = jnp.zeros((NPAD, D), f32).at[:N].set(x_source)
    xt = jnp.zeros((NPAD, D), f32).at[:N].set(x_target)

    def prep(ei):
        src = ei[0].astype(jnp.int32)
        dst = ei[1].astype(jnp.int32)
        pad = EPAD - E
        src = jnp.concatenate([src, jnp.zeros((pad,), jnp.int32)])
        dst = jnp.concatenate([dst, jnp.full((pad,), N, jnp.int32)])
        return src.reshape(NW, G, GSZ), dst.reshape(NW, G, GSZ)

    src2t, dst2t = prep(edge_index_s2t)
    srct2s, dstt2s = prep(edge_index_t2s)
    zeros = jnp.zeros((NPAD, H), f32)
    ones = jnp.ones((GSZ, H), f32)

    agg_c = _make_agg(True)
    agg = _make_agg(False)

    Ps, Pt = _proj0(xs, xt, params_s2t[0][0], params_t2s[0][0])
    accT, cntT = agg_c(Ps, src2t, dst2t, zeros, ones)
    accS, cntS = agg_c(Pt, srct2s, dstt2s, zeros, ones)

    xT_cur, xS_cur = xt, xs
    for i in range(2):
        Wr1, b1 = params_s2t[i][1], params_s2t[i][2]
        Wr2, b2 = params_t2s[i][1], params_t2s[i][2]
        Wl1n, Wl2n = params_s2t[i + 1][0], params_t2s[i + 1][0]
        din = D if i == 0 else H
        outT, outS, PnS, PnT = _make_combine(din)(
            accT, accS, cntT, cntS, xT_cur, xS_cur,
            Wr1, b1.reshape(1, H), Wr2, b2.reshape(1, H), Wl1n, Wl2n)
        accT = agg(PnS, src2t, dst2t, zeros)
        accS = agg(PnT, srct2s, dstt2s, zeros)
        xT_cur, xS_cur = outT, outS

    Wr1, b1 = params_s2t[2][1], params_s2t[2][2]
    Wr2, b2 = params_t2s[2][1], params_t2s[2][2]
    return _make_final(H)(
        accT, accS, cntT, cntS, xT_cur, xS_cur,
        Wr1, b1.reshape(1, H), Wr2, b2.reshape(1, H),
        lin_W.reshape(1, H), lin_b.reshape(1, 1))


# speedup vs baseline: 1.1900x; 1.1900x over previous
import functools

import jax
import jax.numpy as jnp
from jax import lax
from jax.experimental import pallas as pl
from jax.experimental.pallas import tpu as pltpu
from jax.experimental.pallas import tpu_sc as plsc

N = 10000
D = 128
H = 16
NPAD = 10240
E = 320000
GSZ = 128
NW = 32
G = 80
EPW = G * GSZ
EPAD = NW * EPW
RPT = NPAD // 16
GRID = NPAD // RPT

_MESH = plsc.VectorSubcoreMesh(core_axis_name="c", subcore_axis_name="s")


def _agg_body(with_counts, CB, NC, *args):
    if with_counts:
        (p, src_hbm, dst_hbm, zeros_hbm, ones_hbm,
         out_hbm, cnt_hbm,
         src_v, dst_v, rows_v, ones_v, acc_sh, stage_sh, cacc_sh, sem) = args
    else:
        (p, src_hbm, dst_hbm, zeros_hbm,
         out_hbm,
         src_v, dst_v, rows_v, acc_sh, stage_sh, sem) = args
    c = lax.axis_index("c")
    s = lax.axis_index("s")
    wid = c * 16 + s
    r0 = s * RPT
    pltpu.sync_copy(p.at[pl.ds(r0, RPT)], stage_sh.at[pl.ds(r0, RPT)])
    pltpu.sync_copy(zeros_hbm.at[pl.ds(r0, RPT)], acc_sh.at[pl.ds(r0, RPT)])
    if with_counts:
        pltpu.sync_copy(zeros_hbm.at[pl.ds(r0, RPT)], cacc_sh.at[pl.ds(r0, RPT)])
        pltpu.sync_copy(ones_hbm, ones_v)
    pltpu.sync_copy(src_hbm.at[wid], src_v)
    pltpu.sync_copy(dst_hbm.at[wid], dst_v)
    plsc.subcore_barrier()

    def chunk(ck, carry):
        gathers = []
        for b in range(CB):
            gathers.append((b, pltpu.async_copy(
                stage_sh.at[src_v.at[ck * CB + b]],
                rows_v.at[b], sem.at[b])))
        scatters = []
        for b, desc in gathers:
            desc.wait()
            g = ck * CB + b
            scatters.append(pltpu.async_copy(
                rows_v.at[b], acc_sh.at[dst_v.at[g]], sem.at[b], add=True))
            if with_counts:
                scatters.append(pltpu.async_copy(
                    ones_v, cacc_sh.at[dst_v.at[g]], sem.at[b], add=True))
        for desc in scatters:
            desc.wait()
        return carry

    lax.fori_loop(0, NC, chunk, 0)
    plsc.subcore_barrier()
    pltpu.sync_copy(acc_sh.at[pl.ds(r0, RPT)], out_hbm.at[c, pl.ds(r0, RPT)])
    if with_counts:
        pltpu.sync_copy(cacc_sh.at[pl.ds(r0, RPT)], cnt_hbm.at[c, pl.ds(r0, RPT)])


def _make_agg(with_counts):
    CB = 16
    NC = G // CB
    acc_t = jax.ShapeDtypeStruct((2, NPAD, H), jnp.float32)
    cnt_t = jax.ShapeDtypeStruct((2, NPAD, H), jnp.float32)
    out_types = [acc_t, cnt_t] if with_counts else [acc_t]
    scratch = [
        pltpu.VMEM((G, GSZ), jnp.int32),
        pltpu.VMEM((G, GSZ), jnp.int32),
        pltpu.VMEM((CB, GSZ, H), jnp.float32),
    ]
    if with_counts:
        scratch.append(pltpu.VMEM((GSZ, H), jnp.float32))
    scratch += [pltpu.VMEM_SHARED((NPAD, H), jnp.float32)] * 2
    if with_counts:
        scratch.append(pltpu.VMEM_SHARED((NPAD, H), jnp.float32))
    scratch.append(pltpu.SemaphoreType.DMA((CB,)))
    return pl.kernel(
        functools.partial(_agg_body, with_counts, CB, NC),
        out_type=tuple(out_types) if with_counts else out_types[0],
        mesh=_MESH,
        scratch_types=scratch,
        compiler_params=pltpu.CompilerParams(use_tc_tiling_on_sc=False),
    )


def _proj0_body(x1, x2, w1, w2, o1, o2):
    o1[...] = jnp.dot(x1[...], w1[...], preferred_element_type=jnp.float32)
    o2[...] = jnp.dot(x2[...], w2[...], preferred_element_type=jnp.float32)


def _proj0(x1, x2, w1, w2):
    return pl.pallas_call(
        _proj0_body,
        grid=(GRID,),
        in_specs=[
            pl.BlockSpec((RPT, D), lambda i: (i, 0)),
            pl.BlockSpec((RPT, D), lambda i: (i, 0)),
            pl.BlockSpec((D, H), lambda i: (0, 0)),
            pl.BlockSpec((D, H), lambda i: (0, 0)),
        ],
        out_specs=[pl.BlockSpec((RPT, H), lambda i: (i, 0))] * 2,
        out_shape=[jax.ShapeDtypeStruct((NPAD, H), jnp.float32)] * 2,
    )(x1, x2, w1, w2)


def _sage_out(acc, cnt, x, wr, b):
    agg = (acc[0] + acc[1]) / jnp.maximum(cnt[0] + cnt[1], 1.0)
    return agg + jnp.dot(x[...], wr[...], preferred_element_type=jnp.float32) + b[...]


def _combine_body(accT, accS, cntT, cntS, xT, xS, wr1, b1, wr2, b2,
                  wl1n, wl2n, outT, outS, pnS, pnT):
    oT = _sage_out(accT, cntT, xT, wr1, b1)
    outT[...] = oT
    pnT[...] = jnp.dot(oT, wl2n[...], preferred_element_type=jnp.float32)
    oS = _sage_out(accS, cntS, xS, wr2, b2)
    outS[...] = oS
    pnS[...] = jnp.dot(oS, wl1n[...], preferred_element_type=jnp.float32)


def _make_combine(din):
    acc_spec = pl.BlockSpec((2, RPT, H), lambda i: (0, i, 0))
    cnt_spec = pl.BlockSpec((2, RPT, H), lambda i: (0, i, 0))
    w_spec = pl.BlockSpec((din, H), lambda i: (0, 0))
    b_spec = pl.BlockSpec((1, H), lambda i: (0, 0))
    h_spec = pl.BlockSpec((H, H), lambda i: (0, 0))
    row_spec = pl.BlockSpec((RPT, H), lambda i: (i, 0))
    return pl.pallas_call(
        _combine_body,
        grid=(GRID,),
        in_specs=[acc_spec, acc_spec, cnt_spec, cnt_spec,
                  pl.BlockSpec((RPT, din), lambda i: (i, 0)),
                  pl.BlockSpec((RPT, din), lambda i: (i, 0)),
                  w_spec, b_spec, w_spec, b_spec, h_spec, h_spec],
        out_specs=[row_spec] * 4,
        out_shape=[jax.ShapeDtypeStruct((NPAD, H), jnp.float32)] * 4,
    )


def _final_body(accT, accS, cntT, cntS, xT, xS, wr1, b1, wr2, b2,
                lw, lb, out, accum):
    i = pl.program_id(0)

    @pl.when(i == 0)
    def _():
        accum[...] = jnp.zeros_like(accum)

    oT = _sage_out(accT, cntT, xT, wr1, b1)
    oS = _sage_out(accS, cntS, xS, wr2, b2)
    rows = i * RPT + lax.broadcasted_iota(jnp.int32, (RPT, 1), 0)
    sblk = jnp.sum(jnp.where(rows < N, oT + oS, 0.0), axis=0, keepdims=True)
    accum[...] += sblk

    @pl.when(i == GRID - 1)
    def _():
        out[...] = (jnp.sum(accum[...] * lw[...], axis=1, keepdims=True)
                    / (2.0 * N) + lb[...])


def _make_final(din):
    acc_spec = pl.BlockSpec((2, RPT, H), lambda i: (0, i, 0))
    cnt_spec = pl.BlockSpec((2, RPT, H), lambda i: (0, i, 0))
    w_spec = pl.BlockSpec((din, H), lambda i: (0, 0))
    b_spec = pl.BlockSpec((1, H), lambda i: (0, 0))
    return pl.pallas_call(
        _final_body,
        grid=(GRID,),
        in_specs=[acc_spec, acc_spec, cnt_spec, cnt_spec,
                  pl.BlockSpec((RPT, din), lambda i: (i, 0)),
                  pl.BlockSpec((RPT, din), lambda i: (i, 0)),
                  w_spec, b_spec, w_spec, b_spec,
                  pl.BlockSpec((1, H), lambda i: (0, 0)),
                  pl.BlockSpec((1, 1), lambda i: (0, 0))],
        out_specs=pl.BlockSpec((1, 1), lambda i: (0, 0)),
        out_shape=jax.ShapeDtypeStruct((1, 1), jnp.float32),
        scratch_shapes=[pltpu.VMEM((1, H), jnp.float32)],
    )


def kernel(x_source, x_target, edge_index_s2t, edge_index_t2s,
           edge_attr_s2t, edge_attr_t2s, params_s2t, params_t2s,
           lin_W, lin_b):
    f32 = jnp.float32
    xs = jnp.zeros((NPAD, D), f32).at[:N].set(x_source)
    xt = jnp.zeros((NPAD, D), f32).at[:N].set(x_target)

    def prep(ei):
        src = ei[0].astype(jnp.int32)
        dst = ei[1].astype(jnp.int32)
        pad = EPAD - E
        src = jnp.concatenate([src, jnp.zeros((pad,), jnp.int32)])
        dst = jnp.concatenate([dst, jnp.full((pad,), N, jnp.int32)])
        return src.reshape(NW, G, GSZ), dst.reshape(NW, G, GSZ)

    src2t, dst2t = prep(edge_index_s2t)
    srct2s, dstt2s = prep(edge_index_t2s)
    zeros = jnp.zeros((NPAD, H), f32)
    ones = jnp.ones((GSZ, H), f32)

    agg_c = _make_agg(True)
    agg = _make_agg(False)

    Ps, Pt = _proj0(xs, xt, params_s2t[0][0], params_t2s[0][0])
    accT, cntT = agg_c(Ps, src2t, dst2t, zeros, ones)
    accS, cntS = agg_c(Pt, srct2s, dstt2s, zeros, ones)

    xT_cur, xS_cur = xt, xs
    for i in range(2):
        Wr1, b1 = params_s2t[i][1], params_s2t[i][2]
        Wr2, b2 = params_t2s[i][1], params_t2s[i][2]
        Wl1n, Wl2n = params_s2t[i + 1][0], params_t2s[i + 1][0]
        din = D if i == 0 else H
        outT, outS, PnS, PnT = _make_combine(din)(
            accT, accS, cntT, cntS, xT_cur, xS_cur,
            Wr1, b1.reshape(1, H), Wr2, b2.reshape(1, H), Wl1n, Wl2n)
        accT = agg(PnS, src2t, dst2t, zeros)
        accS = agg(PnT, srct2s, dstt2s, zeros)
        xT_cur, xS_cur = outT, outS

    Wr1, b1 = params_s2t[2][1], params_s2t[2][2]
    Wr2, b2 = params_t2s[2][1], params_t2s[2][2]
    return _make_final(H)(
        accT, accS, cntT, cntS, xT_cur, xS_cur,
        Wr1, b1.reshape(1, H), Wr2, b2.reshape(1, H),
        lin_W.reshape(1, H), lin_b.reshape(1, 1))
